# baseline (speedup 1.0000x reference)
import functools

import jax
import jax.numpy as jnp
from jax import lax
from jax.experimental import pallas as pl
from jax.experimental.pallas import tpu as pltpu
from jax.experimental.pallas import tpu_sc as plsc

BATCH = 16384
D = 32
NC = 2
NS = 16
NW = NC * NS
BPW = BATCH // NW

_mesh = plsc.VectorSubcoreMesh(core_axis_name="c", subcore_axis_name="s")


@functools.partial(
    pl.kernel,
    mesh=_mesh,
    compiler_params=pltpu.CompilerParams(needs_layout_passes=False,
                                         use_tc_tiling_on_sc=False),
    out_type=jax.ShapeDtypeStruct((BATCH,), jnp.float32),
    scratch_types=[
        pltpu.VMEM((BPW,), jnp.int32),
        pltpu.VMEM((BPW,), jnp.int32),
        pltpu.VMEM((D, BPW), jnp.float32),
        pltpu.VMEM((D, BPW), jnp.float32),
        pltpu.VMEM((BPW,), jnp.float32),
        pltpu.SemaphoreType.DMA,
        pltpu.SemaphoreType.DMA,
    ],
)
def _mf_kernel(uidx_hbm, midx_hbm, utab_hbm, mtab_hbm, out_hbm,
               uidx_v, midx_v, ucols_v, mcols_v, out_v, sem_u, sem_m):
    wid = lax.axis_index("s") * NC + lax.axis_index("c")
    base = wid * BPW

    pltpu.sync_copy(uidx_hbm.at[pl.ds(base, BPW)], uidx_v)
    pltpu.sync_copy(midx_hbm.at[pl.ds(base, BPW)], midx_v)

    for d in range(D):
        pltpu.async_copy(utab_hbm.at[d].at[uidx_v], ucols_v.at[d], sem_u)
        pltpu.async_copy(mtab_hbm.at[d].at[midx_v], mcols_v.at[d], sem_m)
    for d in range(D):
        pltpu.make_async_copy(utab_hbm.at[d].at[uidx_v],
                              ucols_v.at[d], sem_u).wait()
        pltpu.make_async_copy(mtab_hbm.at[d].at[midx_v],
                              mcols_v.at[d], sem_m).wait()

    def dot_body(g, carry):
        col = pl.ds(g * 16, 16)
        acc = ucols_v[0, col] * mcols_v[0, col]
        for d in range(1, D):
            acc = acc + ucols_v[d, col] * mcols_v[d, col]
        out_v[col] = acc
        return carry

    lax.fori_loop(0, BPW // 16, dot_body, 0)

    pltpu.sync_copy(out_v, out_hbm.at[pl.ds(base, BPW)])


def kernel(user_idx, movie_idx, user_table, movie_table):
    return _mf_kernel(user_idx, movie_idx, user_table.T, movie_table.T)


# speedup vs baseline: 5.6004x; 5.6004x over previous
import functools

import jax
import jax.numpy as jnp
from jax import lax
from jax.experimental import pallas as pl
from jax.experimental.pallas import tpu as pltpu
from jax.experimental.pallas import tpu_sc as plsc

BATCH = 16384
D = 32
NC = 2
NS = 16
NW = NC * NS
BPW = BATCH // NW
RPB = 128 // D
HCH = BPW // 2

_mesh = plsc.VectorSubcoreMesh(core_axis_name="c", subcore_axis_name="s")


@functools.partial(
    pl.kernel,
    mesh=_mesh,
    compiler_params=pltpu.CompilerParams(needs_layout_passes=False,
                                         use_tc_tiling_on_sc=True),
    out_type=jax.ShapeDtypeStruct((BATCH,), jnp.float32),
    scratch_types=[
        pltpu.VMEM((BPW,), jnp.int32),
        pltpu.VMEM((BPW,), jnp.int32),
        pltpu.VMEM((BPW,), jnp.int32),
        pltpu.VMEM((BPW,), jnp.int32),
        pltpu.VMEM((HCH, 128), jnp.float32),
        pltpu.VMEM((HCH, 128), jnp.float32),
        pltpu.VMEM((BPW,), jnp.float32),
        pltpu.SemaphoreType.DMA,
        pltpu.SemaphoreType.DMA,
    ],
)
def _mf_kernel(uidx_hbm, midx_hbm, utab_hbm, mtab_hbm, out_hbm,
               uidx_v, midx_v, urow_v, mrow_v, ubuf_v, mbuf_v, out_v,
               sem_u, sem_m):
    wid = lax.axis_index("s") * NC + lax.axis_index("c")
    base = wid * BPW

    pltpu.sync_copy(uidx_hbm.at[pl.ds(base, BPW)], uidx_v)
    pltpu.sync_copy(midx_hbm.at[pl.ds(base, BPW)], midx_v)

    def rid_body(i, carry):
        sl = pl.ds(i * 16, 16)
        urow_v[sl] = lax.shift_right_logical(uidx_v[sl], 2)
        mrow_v[sl] = lax.shift_right_logical(midx_v[sl], 2)
        return carry

    lax.fori_loop(0, BPW // 16, rid_body, 0)

    lanes = lax.iota(jnp.int32, 16)

    for h in range(2):
        hsl = pl.ds(h * HCH, HCH)
        pltpu.async_copy(utab_hbm.at[urow_v.at[hsl]], ubuf_v, sem_u)
        pltpu.async_copy(mtab_hbm.at[mrow_v.at[hsl]], mbuf_v, sem_m)
        pltpu.make_async_copy(utab_hbm.at[urow_v.at[hsl]], ubuf_v, sem_u).wait()
        pltpu.make_async_copy(mtab_hbm.at[mrow_v.at[hsl]], mbuf_v, sem_m).wait()

        def dot_body(g, carry):
            gsl = pl.ds(h * HCH + g * 16, 16)
            slots = g * 16 + lanes
            uoff = (uidx_v[gsl] & (RPB - 1)) * D
            moff = (midx_v[gsl] & (RPB - 1)) * D
            acc = (plsc.load_gather(ubuf_v, [slots, uoff])
                   * plsc.load_gather(mbuf_v, [slots, moff]))
            for d in range(1, D):
                acc = acc + (plsc.load_gather(ubuf_v, [slots, uoff + d])
                             * plsc.load_gather(mbuf_v, [slots, moff + d]))
            out_v[gsl] = acc
            return carry

        lax.fori_loop(0, HCH // 16, dot_body, 0)

    pltpu.sync_copy(out_v, out_hbm.at[pl.ds(base, BPW)])


def kernel(user_idx, movie_idx, user_table, movie_table):
    utab = user_table.reshape(1000000 // RPB, 128)
    mtab = movie_table.reshape(1000000 // RPB, 128)
    return _mf_kernel(user_idx, movie_idx, utab, mtab)
